# baseline (speedup 1.0000x reference)
import functools

import jax
import jax.numpy as jnp
from jax import lax
from jax.experimental import pallas as pl
from jax.experimental.pallas import tpu as pltpu
from jax.experimental.pallas import tpu_sc as plsc

N = 10000
E = 320000
D = 128
NC = 2
NS = 16
NW = NC * NS
C = 128
K = 80
G = 16
KG = K // G
EW_PAD = K * C
E_PAD = EW_PAD * NW
N_ACC = 10112
RPS_A = N_ACC // NS

_HI = lax.Precision.HIGHEST
_mesh = plsc.VectorSubcoreMesh(core_axis_name="c", subcore_axis_name="s")


@functools.partial(
    pl.kernel,
    out_type=jax.ShapeDtypeStruct((NC, N_ACC, 16), jnp.float32),
    mesh=_mesh,
    scratch_types=[
        pltpu.VMEM((K, C), jnp.int32),
        pltpu.VMEM((C, 16), jnp.float32),
        pltpu.VMEM_SHARED((N_ACC, 16), jnp.float32),
    ],
)
def _sc_degree(col_hbm, zeros_hbm, out_hbm, col_v, ones_v, acc_sh):
    cid = lax.axis_index("c")
    sid = lax.axis_index("s")
    w = cid * NS + sid

    def setones(i, carry):
        ones_v[i] = jnp.ones((16,), jnp.float32)
        return carry

    lax.fori_loop(0, C, setones, 0)
    pltpu.sync_copy(zeros_hbm.at[pl.ds(sid * RPS_A, RPS_A)],
                    acc_sh.at[pl.ds(sid * RPS_A, RPS_A)])
    pltpu.sync_copy(col_hbm.at[w], col_v)
    plsc.subcore_barrier()

    def chunk(j, carry):
        pltpu.sync_copy(ones_v, acc_sh.at[col_v.at[j]], add=True)
        return carry

    lax.fori_loop(0, K, chunk, 0)
    plsc.subcore_barrier()
    pltpu.sync_copy(acc_sh.at[pl.ds(sid * RPS_A, RPS_A)],
                    out_hbm.at[cid, pl.ds(sid * RPS_A, RPS_A)])


@functools.partial(
    pl.kernel,
    out_type=jax.ShapeDtypeStruct((NC, N_ACC, D), jnp.float32),
    mesh=_mesh,
    scratch_types=[
        pltpu.VMEM((2, G, C), jnp.int32),
        pltpu.VMEM((2, G, C), jnp.int32),
        pltpu.VMEM((C, D), jnp.float32),
        pltpu.VMEM((C, D), jnp.float32),
        pltpu.VMEM_SHARED((N_ACC, D), jnp.float32),
        pltpu.SemaphoreType.DMA,
        pltpu.SemaphoreType.DMA,
        pltpu.SemaphoreType.DMA,
    ],
)
def _sc_segment_sum(y_hbm, idx_hbm, zeros_hbm, out_hbm,
                    idx_a, idx_b, buf0, buf1, acc_sh, sem_i, sem0, sem1):
    cid = lax.axis_index("c")
    sid = lax.axis_index("s")
    w = cid * NS + sid

    pltpu.sync_copy(zeros_hbm.at[pl.ds(sid * RPS_A, RPS_A)],
                    acc_sh.at[pl.ds(sid * RPS_A, RPS_A)])
    ibufs = (idx_a, idx_b)
    bufs = (buf0, buf1)
    sems = (sem0, sem1)
    pltpu.sync_copy(idx_hbm.at[w, 0], idx_a)
    plsc.subcore_barrier()

    if KG > 1:
        pltpu.async_copy(idx_hbm.at[w, 1], idx_b, sem_i)
    pltpu.async_copy(y_hbm.at[idx_a.at[0, 0]], buf0, sem0)
    for c in range(K):
        g, jj = divmod(c, G)
        ib = ibufs[g % 2]
        if jj == 0 and 1 <= g and g + 1 < KG:
            pltpu.async_copy(idx_hbm.at[w, g + 1], ibufs[(g + 1) % 2], sem_i)
        pltpu.make_async_copy(y_hbm.at[ib.at[0, jj]], bufs[c % 2],
                              sems[c % 2]).wait()
        if c + 1 < K:
            g1, jj1 = divmod(c + 1, G)
            ib1 = ibufs[g1 % 2]
            if jj1 == 0:
                pltpu.make_async_copy(idx_hbm.at[w, g1], ib1, sem_i).wait()
            pltpu.async_copy(y_hbm.at[ib1.at[0, jj1]], bufs[(c + 1) % 2],
                             sems[(c + 1) % 2])
        pltpu.sync_copy(bufs[c % 2], acc_sh.at[ib.at[1, jj]], add=True)
    plsc.subcore_barrier()
    pltpu.sync_copy(acc_sh.at[pl.ds(sid * RPS_A, RPS_A)],
                    out_hbm.at[cid, pl.ds(sid * RPS_A, RPS_A)])


_BLK = 512
_GRID = (pl.cdiv(N, _BLK),)


def _rows(shape):
    return pl.BlockSpec((_BLK,) + shape[1:], lambda i: (i,) + (0,) * (len(shape) - 1))


def _full(shape):
    return pl.BlockSpec(shape, lambda i: (0,) * len(shape))


def _tc_encode_body(x_r, w1_r, b1_r, w2_r, b2_r, w0_r, dp0_r, dp1_r,
                    y_r, dinv_r):
    h = jnp.maximum(jnp.dot(x_r[...], w1_r[...], precision=_HI) + b1_r[...], 0.0)
    h = jnp.maximum(jnp.dot(h, w2_r[...], precision=_HI) + b2_r[...], 0.0)
    xw = jnp.dot(h, w0_r[...], precision=_HI)
    deg = dp0_r[:, :1] + dp1_r[:, :1] + 1.0
    dinv = lax.rsqrt(deg)
    y_r[...] = xw * dinv
    dinv_r[...] = jnp.broadcast_to(dinv, (_BLK, D))


def _tc_encode(x, w1, b1, w2, b2, w0, dp0, dp1):
    return pl.pallas_call(
        _tc_encode_body,
        grid=_GRID,
        in_specs=[_rows(x.shape), _full(w1.shape), _full(b1.shape),
                  _full(w2.shape), _full(b2.shape), _full(w0.shape),
                  _rows(dp0.shape), _rows(dp1.shape)],
        out_specs=[_rows((N, D)), _rows((N, D))],
        out_shape=[jax.ShapeDtypeStruct((N, D), jnp.float32),
                   jax.ShapeDtypeStruct((N, D), jnp.float32)],
    )(x, w1, b1, w2, b2, w0, dp0, dp1)


def _tc_mid_body(p0_r, p1_r, y_r, dinv_r, b_r, w_r, out_r):
    h = jnp.maximum((p0_r[...] + p1_r[...] + y_r[...]) * dinv_r[...] + b_r[...], 0.0)
    out_r[...] = jnp.dot(h, w_r[...], precision=_HI) * dinv_r[...]


def _tc_mid(p0, p1, y, dinvb, b, w):
    return pl.pallas_call(
        _tc_mid_body,
        grid=_GRID,
        in_specs=[_rows(p0.shape), _rows(p1.shape), _rows(y.shape),
                  _rows(dinvb.shape), _full(b.shape), _full(w.shape)],
        out_specs=_rows((N, D)),
        out_shape=jax.ShapeDtypeStruct((N, D), jnp.float32),
    )(p0, p1, y, dinvb, b, w)


def _tc_decode_body(q0_r, q1_r, y_r, dinv_r, b_r, w1_r, b1_r, w2_r, b2_r,
                    out_r):
    h = jnp.maximum((q0_r[...] + q1_r[...] + y_r[...]) * dinv_r[...] + b_r[...], 0.0)
    h = jnp.maximum(jnp.dot(h, w1_r[...], precision=_HI) + b1_r[...], 0.0)
    z = jnp.dot(h, w2_r[...], precision=_HI) + b2_r[...]
    out_r[...] = 1.0 / (1.0 + jnp.exp(-z))


def _tc_decode(q0, q1, y, dinvb, b, w1, b1, w2, b2):
    nout = w2.shape[1]
    return pl.pallas_call(
        _tc_decode_body,
        grid=_GRID,
        in_specs=[_rows(q0.shape), _rows(q1.shape), _rows(y.shape),
                  _rows(dinvb.shape), _full(b.shape), _full(w1.shape),
                  _full(b1.shape), _full(w2.shape), _full(b2.shape)],
        out_specs=_rows((N, nout)),
        out_shape=jax.ShapeDtypeStruct((N, nout), jnp.float32),
    )(q0, q1, y, dinvb, b, w1, b1, w2, b2)


def kernel(x, edge_index, edge_attr, enc_W1, enc_b1, enc_W2, enc_b2,
           gnn_W0, gnn_b0, gnn_W1, gnn_b1, dec_W1, dec_b1, dec_W2, dec_b2):
    del edge_attr
    pad = E_PAD - E
    row_p = jnp.concatenate(
        [edge_index[0], jnp.zeros((pad,), jnp.int32)]).reshape(NW, KG, G, C)
    col_p = jnp.concatenate(
        [edge_index[1], jnp.full((pad,), N, jnp.int32)]).reshape(NW, KG, G, C)
    idx_p = jnp.stack([row_p, col_p], axis=2)
    zeros16 = jnp.zeros((N_ACC, 16), jnp.float32)
    zerosD = jnp.zeros((N_ACC, D), jnp.float32)

    degp = _sc_degree(col_p.reshape(NW, K, C), zeros16)
    y0, dinvb = _tc_encode(x, enc_W1, enc_b1.reshape(1, -1), enc_W2,
                           enc_b2.reshape(1, -1), gnn_W0, degp[0], degp[1])
    p = _sc_segment_sum(y0, idx_p, zerosD)
    y1 = _tc_mid(p[0, :N], p[1, :N], y0, dinvb, gnn_b0.reshape(1, -1), gnn_W1)
    q = _sc_segment_sum(y1, idx_p, zerosD)
    out = _tc_decode(q[0, :N], q[1, :N], y1, dinvb, gnn_b1.reshape(1, -1), dec_W1,
                     dec_b1.reshape(1, -1), dec_W2, dec_b2.reshape(1, -1))
    return out


# speedup vs baseline: 1.0011x; 1.0011x over previous
import functools

import jax
import jax.numpy as jnp
from jax import lax
from jax.experimental import pallas as pl
from jax.experimental.pallas import tpu as pltpu
from jax.experimental.pallas import tpu_sc as plsc

N = 10000
E = 320000
D = 128
NC = 2
NS = 16
NW = NC * NS
C = 128
K = 80
G = 16
KG = K // G
EW_PAD = K * C
E_PAD = EW_PAD * NW
N_ACC = 10112
RPS_A = N_ACC // NS

_HI = lax.Precision.HIGHEST
_mesh = plsc.VectorSubcoreMesh(core_axis_name="c", subcore_axis_name="s")


@functools.partial(
    pl.kernel,
    out_type=jax.ShapeDtypeStruct((NC, N_ACC, 16), jnp.float32),
    mesh=_mesh,
    scratch_types=[
        pltpu.VMEM((K, C), jnp.int32),
        pltpu.VMEM((C, 16), jnp.float32),
        pltpu.VMEM_SHARED((N_ACC, 16), jnp.float32),
    ],
)
def _sc_degree(col_hbm, zeros_hbm, out_hbm, col_v, ones_v, acc_sh):
    cid = lax.axis_index("c")
    sid = lax.axis_index("s")
    w = cid * NS + sid

    def setones(i, carry):
        ones_v[i] = jnp.ones((16,), jnp.float32)
        return carry

    lax.fori_loop(0, C, setones, 0)
    pltpu.sync_copy(zeros_hbm.at[pl.ds(sid * RPS_A, RPS_A)],
                    acc_sh.at[pl.ds(sid * RPS_A, RPS_A)])
    pltpu.sync_copy(col_hbm.at[w], col_v)
    plsc.subcore_barrier()

    def chunk(j, carry):
        pltpu.sync_copy(ones_v, acc_sh.at[col_v.at[j]], add=True)
        return carry

    lax.fori_loop(0, K, chunk, 0)
    plsc.subcore_barrier()
    pltpu.sync_copy(acc_sh.at[pl.ds(sid * RPS_A, RPS_A)],
                    out_hbm.at[cid, pl.ds(sid * RPS_A, RPS_A)])


@functools.partial(
    pl.kernel,
    out_type=jax.ShapeDtypeStruct((NC, N_ACC, D), jnp.float32),
    mesh=_mesh,
    scratch_types=[
        pltpu.VMEM((2, G, C), jnp.int32),
        pltpu.VMEM((2, G, C), jnp.int32),
        pltpu.VMEM((C, D), jnp.float32),
        pltpu.VMEM((C, D), jnp.float32),
        pltpu.VMEM_SHARED((N_ACC, D), jnp.float32),
        pltpu.SemaphoreType.DMA,
        pltpu.SemaphoreType.DMA,
        pltpu.SemaphoreType.DMA,
    ],
)
def _sc_segment_sum(y_hbm, idx_hbm, zeros_hbm, out_hbm,
                    idx_a, idx_b, buf0, buf1, acc_sh, sem_i, sem0, sem1):
    cid = lax.axis_index("c")
    sid = lax.axis_index("s")
    w = cid * NS + sid

    pltpu.sync_copy(zeros_hbm.at[pl.ds(sid * RPS_A, RPS_A)],
                    acc_sh.at[pl.ds(sid * RPS_A, RPS_A)])
    ibufs = (idx_a, idx_b)
    bufs = (buf0, buf1)
    sems = (sem0, sem1)
    pltpu.sync_copy(idx_hbm.at[w, 0], idx_a)
    plsc.subcore_barrier()

    if KG > 1:
        pltpu.async_copy(idx_hbm.at[w, 1], idx_b, sem_i)
    pltpu.async_copy(y_hbm.at[idx_a.at[0, 0]], buf0, sem0)
    for c in range(K):
        g, jj = divmod(c, G)
        ib = ibufs[g % 2]
        if jj == 0 and 1 <= g and g + 1 < KG:
            pltpu.async_copy(idx_hbm.at[w, g + 1], ibufs[(g + 1) % 2], sem_i)
        pltpu.make_async_copy(y_hbm.at[ib.at[0, jj]], bufs[c % 2],
                              sems[c % 2]).wait()
        if c + 1 < K:
            g1, jj1 = divmod(c + 1, G)
            ib1 = ibufs[g1 % 2]
            if jj1 == 0:
                pltpu.make_async_copy(idx_hbm.at[w, g1], ib1, sem_i).wait()
            pltpu.async_copy(y_hbm.at[ib1.at[0, jj1]], bufs[(c + 1) % 2],
                             sems[(c + 1) % 2])
        pltpu.sync_copy(bufs[c % 2], acc_sh.at[ib.at[1, jj]], add=True)
    plsc.subcore_barrier()
    pltpu.sync_copy(acc_sh.at[pl.ds(sid * RPS_A, RPS_A)],
                    out_hbm.at[cid, pl.ds(sid * RPS_A, RPS_A)])


_BLK = 512
_GRID = (pl.cdiv(N, _BLK),)


def _rows(shape):
    return pl.BlockSpec((_BLK,) + shape[1:], lambda i: (i,) + (0,) * (len(shape) - 1))


def _full(shape):
    return pl.BlockSpec(shape, lambda i: (0,) * len(shape))


def _tc_encode_body(x_r, w1_r, b1_r, w2_r, b2_r, w0_r, dp0_r, dp1_r,
                    y_r, dinv_r):
    h = jnp.maximum(jnp.dot(x_r[...], w1_r[...], precision=_HI) + b1_r[...], 0.0)
    h = jnp.maximum(jnp.dot(h, w2_r[...], precision=_HI) + b2_r[...], 0.0)
    xw = jnp.dot(h, w0_r[...], precision=_HI)
    deg = dp0_r[:, :1] + dp1_r[:, :1] + 1.0
    dinv = lax.rsqrt(deg)
    y_r[...] = xw * dinv
    dinv_r[...] = jnp.broadcast_to(dinv, (_BLK, D))


def _tc_encode(x, w1, b1, w2, b2, w0, dp0, dp1):
    return pl.pallas_call(
        _tc_encode_body,
        grid=_GRID,
        in_specs=[_rows(x.shape), _full(w1.shape), _full(b1.shape),
                  _full(w2.shape), _full(b2.shape), _full(w0.shape),
                  _rows(dp0.shape), _rows(dp1.shape)],
        out_specs=[_rows((N, D)), _rows((N, D))],
        out_shape=[jax.ShapeDtypeStruct((N, D), jnp.float32),
                   jax.ShapeDtypeStruct((N, D), jnp.float32)],
    )(x, w1, b1, w2, b2, w0, dp0, dp1)


def _tc_mid_body(p0_r, p1_r, y_r, dinv_r, b_r, w_r, out_r):
    h = jnp.maximum((p0_r[...] + p1_r[...] + y_r[...]) * dinv_r[...] + b_r[...], 0.0)
    out_r[...] = jnp.dot(h, w_r[...], precision=_HI) * dinv_r[...]


def _tc_mid(p0, p1, y, dinvb, b, w):
    return pl.pallas_call(
        _tc_mid_body,
        grid=_GRID,
        in_specs=[_rows(p0.shape), _rows(p1.shape), _rows(y.shape),
                  _rows(dinvb.shape), _full(b.shape), _full(w.shape)],
        out_specs=_rows((N, D)),
        out_shape=jax.ShapeDtypeStruct((N, D), jnp.float32),
    )(p0, p1, y, dinvb, b, w)


def _tc_decode_body(q0_r, q1_r, y_r, dinv_r, b_r, w1_r, b1_r, w2_r, b2_r,
                    out_r):
    h = jnp.maximum((q0_r[...] + q1_r[...] + y_r[...]) * dinv_r[...] + b_r[...], 0.0)
    h = jnp.maximum(jnp.dot(h, w1_r[...], precision=_HI) + b1_r[...], 0.0)
    z = jnp.dot(h, w2_r[...], precision=_HI) + b2_r[...]
    out_r[...] = 1.0 / (1.0 + jnp.exp(-z))


def _tc_decode(q0, q1, y, dinvb, b, w1, b1, w2, b2):
    nout = w2.shape[1]
    return pl.pallas_call(
        _tc_decode_body,
        grid=_GRID,
        in_specs=[_rows(q0.shape), _rows(q1.shape), _rows(y.shape),
                  _rows(dinvb.shape), _full(b.shape), _full(w1.shape),
                  _full(b1.shape), _full(w2.shape), _full(b2.shape)],
        out_specs=_rows((N, nout)),
        out_shape=jax.ShapeDtypeStruct((N, nout), jnp.float32),
    )(q0, q1, y, dinvb, b, w1, b1, w2, b2)


def kernel(x, edge_index, edge_attr, enc_W1, enc_b1, enc_W2, enc_b2,
           gnn_W0, gnn_b0, gnn_W1, gnn_b1, dec_W1, dec_b1, dec_W2, dec_b2):
    del edge_attr
    pad = E_PAD - E
    row_p = jnp.concatenate(
        [edge_index[0], jnp.zeros((pad,), jnp.int32)]).reshape(NW, KG, G, C)
    dummy = N + jnp.arange(pad, dtype=jnp.int32) % (N_ACC - N)
    col_p = jnp.concatenate(
        [edge_index[1], dummy]).reshape(NW, KG, G, C)
    idx_p = jnp.stack([row_p, col_p], axis=2)
    zeros16 = jnp.zeros((N_ACC, 16), jnp.float32)
    zerosD = jnp.zeros((N_ACC, D), jnp.float32)

    degp = _sc_degree(col_p.reshape(NW, K, C), zeros16)
    y0, dinvb = _tc_encode(x, enc_W1, enc_b1.reshape(1, -1), enc_W2,
                           enc_b2.reshape(1, -1), gnn_W0, degp[0], degp[1])
    p = _sc_segment_sum(y0, idx_p, zerosD)
    y1 = _tc_mid(p[0, :N], p[1, :N], y0, dinvb, gnn_b0.reshape(1, -1), gnn_W1)
    q = _sc_segment_sum(y1, idx_p, zerosD)
    out = _tc_decode(q[0, :N], q[1, :N], y1, dinvb, gnn_b1.reshape(1, -1), dec_W1,
                     dec_b1.reshape(1, -1), dec_W2, dec_b2.reshape(1, -1))
    return out


# speedup vs baseline: 2.6898x; 2.6868x over previous
import functools

import jax
import jax.numpy as jnp
from jax import lax
from jax.experimental import pallas as pl
from jax.experimental.pallas import tpu as pltpu
from jax.experimental.pallas import tpu_sc as plsc

N = 10000
E = 320000
D = 128
NC = 2
NS = 16
NW = NC * NS
C = 128
K = 80
G = 16
KG = K // G
EW_PAD = K * C
E_PAD = EW_PAD * NW
N_ACC = 10112
RPS_A = N_ACC // NS

_HI = lax.Precision.HIGHEST
_mesh = plsc.VectorSubcoreMesh(core_axis_name="c", subcore_axis_name="s")


@functools.partial(
    pl.kernel,
    out_type=jax.ShapeDtypeStruct((NC, N_ACC, 16), jnp.float32),
    mesh=_mesh,
    scratch_types=[
        pltpu.VMEM((K, C), jnp.int32),
        pltpu.VMEM((C, 16), jnp.float32),
        pltpu.VMEM_SHARED((N_ACC, 16), jnp.float32),
    ],
)
def _sc_degree(col_hbm, zeros_hbm, out_hbm, col_v, ones_v, acc_sh):
    cid = lax.axis_index("c")
    sid = lax.axis_index("s")
    w = cid * NS + sid

    def setones(i, carry):
        ones_v[i] = jnp.ones((16,), jnp.float32)
        return carry

    lax.fori_loop(0, C, setones, 0)
    pltpu.sync_copy(zeros_hbm.at[pl.ds(sid * RPS_A, RPS_A)],
                    acc_sh.at[pl.ds(sid * RPS_A, RPS_A)])
    pltpu.sync_copy(col_hbm.at[w], col_v)
    plsc.subcore_barrier()

    def chunk(j, carry):
        pltpu.sync_copy(ones_v, acc_sh.at[col_v.at[j]], add=True)
        return carry

    lax.fori_loop(0, K, chunk, 0)
    plsc.subcore_barrier()
    pltpu.sync_copy(acc_sh.at[pl.ds(sid * RPS_A, RPS_A)],
                    out_hbm.at[cid, pl.ds(sid * RPS_A, RPS_A)])


@functools.partial(
    pl.kernel,
    out_type=jax.ShapeDtypeStruct((NC, N_ACC, D), jnp.float32),
    mesh=_mesh,
    scratch_types=[
        pltpu.VMEM((2, G, C), jnp.int32),
        pltpu.VMEM((2, G, C), jnp.int32),
        pltpu.VMEM((C, D), jnp.float32),
        pltpu.VMEM((C, D), jnp.float32),
        pltpu.VMEM_SHARED((N_ACC, D), jnp.float32),
        pltpu.SemaphoreType.DMA,
        pltpu.SemaphoreType.DMA,
        pltpu.SemaphoreType.DMA,
    ],
)
def _sc_segment_sum(y_hbm, idx_hbm, zeros_hbm, out_hbm,
                    idx_a, idx_b, buf0, buf1, acc_sh, sem_i, sem0, sem1):
    cid = lax.axis_index("c")
    sid = lax.axis_index("s")
    w = cid * NS + sid

    pltpu.sync_copy(zeros_hbm.at[pl.ds(sid * RPS_A, RPS_A)],
                    acc_sh.at[pl.ds(sid * RPS_A, RPS_A)])
    ibufs = (idx_a, idx_b)
    bufs = (buf0, buf1)
    sems = (sem0, sem1)
    pltpu.sync_copy(idx_hbm.at[w, 0], idx_a)
    plsc.subcore_barrier()

    if KG > 1:
        pltpu.async_copy(idx_hbm.at[w, 1], idx_b, sem_i)
    pltpu.async_copy(y_hbm.at[idx_a.at[0, 0]], buf0, sem0)
    for c in range(K):
        g, jj = divmod(c, G)
        ib = ibufs[g % 2]
        if jj == 0 and 1 <= g and g + 1 < KG:
            pltpu.async_copy(idx_hbm.at[w, g + 1], ibufs[(g + 1) % 2], sem_i)
        pltpu.make_async_copy(y_hbm.at[ib.at[0, jj]], bufs[c % 2],
                              sems[c % 2]).wait()
        if c + 1 < K:
            g1, jj1 = divmod(c + 1, G)
            ib1 = ibufs[g1 % 2]
            if jj1 == 0:
                pltpu.make_async_copy(idx_hbm.at[w, g1], ib1, sem_i).wait()
            pltpu.async_copy(y_hbm.at[ib1.at[0, jj1]], bufs[(c + 1) % 2],
                             sems[(c + 1) % 2])
        pltpu.sync_copy(bufs[c % 2], acc_sh.at[ib.at[1, jj]], add=True)
    plsc.subcore_barrier()
    pltpu.sync_copy(acc_sh.at[pl.ds(sid * RPS_A, RPS_A)],
                    out_hbm.at[cid, pl.ds(sid * RPS_A, RPS_A)])


_BLK = 512
_GRID = (pl.cdiv(N, _BLK),)


def _rows(shape):
    return pl.BlockSpec((_BLK,) + shape[1:], lambda i: (i,) + (0,) * (len(shape) - 1))


def _full(shape):
    return pl.BlockSpec(shape, lambda i: (0,) * len(shape))


def _tc_encode_body(x_r, w1_r, b1_r, w2_r, b2_r, w0_r, dp0_r, dp1_r,
                    y_r, dinv_r):
    h = jnp.maximum(jnp.dot(x_r[...], w1_r[...], precision=_HI) + b1_r[...], 0.0)
    h = jnp.maximum(jnp.dot(h, w2_r[...], precision=_HI) + b2_r[...], 0.0)
    xw = jnp.dot(h, w0_r[...], precision=_HI)
    deg = dp0_r[:, :1] + dp1_r[:, :1] + 1.0
    dinv = lax.rsqrt(deg)
    y_r[...] = xw * dinv
    dinv_r[...] = jnp.broadcast_to(dinv, (_BLK, D))


def _tc_encode(x, w1, b1, w2, b2, w0, dp0, dp1):
    return pl.pallas_call(
        _tc_encode_body,
        grid=_GRID,
        in_specs=[_rows(x.shape), _full(w1.shape), _full(b1.shape),
                  _full(w2.shape), _full(b2.shape), _full(w0.shape),
                  _rows(dp0.shape), _rows(dp1.shape)],
        out_specs=[_rows((N, D)), _rows((N, D))],
        out_shape=[jax.ShapeDtypeStruct((N, D), jnp.float32),
                   jax.ShapeDtypeStruct((N, D), jnp.float32)],
    )(x, w1, b1, w2, b2, w0, dp0, dp1)


def _tc_mid_body(p0_r, p1_r, y_r, dinv_r, b_r, w_r, out_r):
    h = jnp.maximum((p0_r[...] + p1_r[...] + y_r[...]) * dinv_r[...] + b_r[...], 0.0)
    out_r[...] = jnp.dot(h, w_r[...], precision=_HI) * dinv_r[...]


def _tc_mid(p0, p1, y, dinvb, b, w):
    return pl.pallas_call(
        _tc_mid_body,
        grid=_GRID,
        in_specs=[_rows(p0.shape), _rows(p1.shape), _rows(y.shape),
                  _rows(dinvb.shape), _full(b.shape), _full(w.shape)],
        out_specs=_rows((N, D)),
        out_shape=jax.ShapeDtypeStruct((N, D), jnp.float32),
    )(p0, p1, y, dinvb, b, w)


def _tc_decode_body(q0_r, q1_r, y_r, dinv_r, b_r, w1_r, b1_r, w2_r, b2_r,
                    out_r):
    h = jnp.maximum((q0_r[...] + q1_r[...] + y_r[...]) * dinv_r[...] + b_r[...], 0.0)
    h = jnp.maximum(jnp.dot(h, w1_r[...], precision=_HI) + b1_r[...], 0.0)
    z = jnp.dot(h, w2_r[...], precision=_HI) + b2_r[...]
    out_r[...] = 1.0 / (1.0 + jnp.exp(-z))


def _tc_decode(q0, q1, y, dinvb, b, w1, b1, w2, b2):
    nout = w2.shape[1]
    return pl.pallas_call(
        _tc_decode_body,
        grid=_GRID,
        in_specs=[_rows(q0.shape), _rows(q1.shape), _rows(y.shape),
                  _rows(dinvb.shape), _full(b.shape), _full(w1.shape),
                  _full(b1.shape), _full(w2.shape), _full(b2.shape)],
        out_specs=_rows((N, nout)),
        out_shape=jax.ShapeDtypeStruct((N, nout), jnp.float32),
    )(q0, q1, y, dinvb, b, w1, b1, w2, b2)


def kernel(x, edge_index, edge_attr, enc_W1, enc_b1, enc_W2, enc_b2,
           gnn_W0, gnn_b0, gnn_W1, gnn_b1, dec_W1, dec_b1, dec_W2, dec_b2):
    del edge_attr
    pad = E_PAD - E
    ppw = pad // NW
    rpw = E // NW
    pad_rows = (jnp.arange(pad, dtype=jnp.int32) * 37) % N
    pad_cols = N + jnp.arange(pad, dtype=jnp.int32) % (N_ACC - N)
    row_p = jnp.concatenate(
        [edge_index[0].reshape(NW, rpw), pad_rows.reshape(NW, ppw)],
        axis=1).reshape(NW, KG, G, C)
    col_p = jnp.concatenate(
        [edge_index[1].reshape(NW, rpw), pad_cols.reshape(NW, ppw)],
        axis=1).reshape(NW, KG, G, C)
    idx_p = jnp.stack([row_p, col_p], axis=2)
    zeros16 = jnp.zeros((N_ACC, 16), jnp.float32)
    zerosD = jnp.zeros((N_ACC, D), jnp.float32)

    degp = _sc_degree(col_p.reshape(NW, K, C), zeros16)
    y0, dinvb = _tc_encode(x, enc_W1, enc_b1.reshape(1, -1), enc_W2,
                           enc_b2.reshape(1, -1), gnn_W0, degp[0], degp[1])
    p = _sc_segment_sum(y0, idx_p, zerosD)
    y1 = _tc_mid(p[0, :N], p[1, :N], y0, dinvb, gnn_b0.reshape(1, -1), gnn_W1)
    q = _sc_segment_sum(y1, idx_p, zerosD)
    out = _tc_decode(q[0, :N], q[1, :N], y1, dinvb, gnn_b1.reshape(1, -1), dec_W1,
                     dec_b1.reshape(1, -1), dec_W2, dec_b2.reshape(1, -1))
    return out
